# baseline (speedup 1.0000x reference)
import functools

import jax
import jax.numpy as jnp
from jax.experimental import pallas as pl
from jax.experimental.pallas import tpu as pltpu

_N = 10000
_BM = 400


def _support_body(x_ref, w_ref, out_ref):
    out_ref[...] = jnp.dot(
        x_ref[...], w_ref[...], preferred_element_type=jnp.float32
    )


def _spmm_body(adj_ref, support_ref, out_ref):
    out_ref[...] = jnp.dot(
        adj_ref[...].astype(jnp.bfloat16),
        support_ref[...].astype(jnp.bfloat16),
        preferred_element_type=jnp.float32,
    )


@functools.partial(jax.jit, static_argnames=())
def kernel(input, adj, W):
    n, in_f = input.shape
    out_f = W.shape[1]
    support = pl.pallas_call(
        _support_body,
        out_shape=jax.ShapeDtypeStruct((n, out_f), jnp.float32),
    )(input, W)
    return pl.pallas_call(
        _spmm_body,
        grid=(n // _BM,),
        in_specs=[
            pl.BlockSpec((_BM, n), lambda m: (m, 0)),
            pl.BlockSpec((n, out_f), lambda m: (0, 0)),
        ],
        out_specs=pl.BlockSpec((_BM, out_f), lambda m: (m, 0)),
        out_shape=jax.ShapeDtypeStruct((n, out_f), jnp.float32),
        compiler_params=pltpu.CompilerParams(
            dimension_semantics=("parallel",),
        ),
    )(adj, support)
